# baseline (speedup 1.0000x reference)
import functools

import jax
import jax.numpy as jnp
from jax import lax
from jax.experimental import pallas as pl
from jax.experimental.pallas import tpu as pltpu
from jax.experimental.pallas import tpu_sc as plsc

B = 1024
L = 50
E = 64
H = 64
NW = 32
N_IDX = B * L
B_PER_W = N_IDX // NW
CHUNK = 80
NCHUNK = B_PER_W // CHUNK
NPASS = 2
CPP = NCHUNK // NPASS

T_BLK = 2
N_BLK = L // T_BLK


def _sc_gather_body(table_hbm, idx_hbm, out_hbm, idx_v, rows_v, sem):
    wid = lax.axis_index("s") * 2 + lax.axis_index("c")
    base = wid * B_PER_W
    pltpu.sync_copy(idx_hbm.at[wid], idx_v)
    for p in range(NPASS):
        copies = []
        for j in range(CPP):
            copies.append(
                pltpu.async_copy(
                    table_hbm.at[idx_v.at[p * CPP + j]],
                    rows_v.at[pl.ds(j * CHUNK, CHUNK)],
                    sem,
                )
            )
        for cp in copies:
            cp.wait()
        pltpu.sync_copy(
            rows_v, out_hbm.at[pl.ds(base + p * CPP * CHUNK, CPP * CHUNK)]
        )


def _sc_gather(table, idx3):
    kern = functools.partial(
        pl.kernel,
        mesh=plsc.VectorSubcoreMesh(core_axis_name="c", subcore_axis_name="s"),
        out_type=jax.ShapeDtypeStruct((N_IDX, 2 * E), jnp.float32),
        scratch_types=[
            pltpu.VMEM((NCHUNK, CHUNK), jnp.int32),
            pltpu.VMEM((CPP * CHUNK, 2 * E), jnp.float32),
            pltpu.SemaphoreType.DMA,
        ],
    )(_sc_gather_body)
    return kern(table, idx3)


def _lstm_body(x_ref, par_ref, w_ref, b_ref, h0_ref, c0_ref,
               ys_ref, h_ref, c_ref):
    i = pl.program_id(0)

    @pl.when(i == 0)
    def _():
        h_ref[...] = h0_ref[...]
        c_ref[...] = c0_ref[...]

    h = h_ref[...]
    c = c_ref[...]
    b = b_ref[...]
    for j in range(T_BLK):
        left = x_ref[:, j * 2 * E:j * 2 * E + E]
        right = x_ref[:, j * 2 * E + E:(j + 1) * 2 * E]
        p = par_ref[:, j * E:(j + 1) * E] != 0
        x_t = jnp.where(p, right, left)
        xh = jnp.concatenate([x_t, h], axis=1)
        gates = jnp.dot(xh, w_ref[...], preferred_element_type=jnp.float32) + b
        ig = jax.nn.sigmoid(gates[:, 0:H])
        fg = jax.nn.sigmoid(gates[:, H:2 * H])
        gg = jnp.tanh(gates[:, 2 * H:3 * H])
        og = jax.nn.sigmoid(gates[:, 3 * H:4 * H])
        c = fg * c + ig * gg
        h = og * jnp.tanh(c)
        ys_ref[:, j * H:(j + 1) * H] = h
    h_ref[...] = h
    c_ref[...] = c


def _lstm(x2d, par2d, w_cat, bias, h0, c0, interpret=False):
    return pl.pallas_call(
        _lstm_body,
        grid=(N_BLK,),
        in_specs=[
            pl.BlockSpec((B, T_BLK * 2 * E), lambda i: (0, i)),
            pl.BlockSpec((B, T_BLK * E), lambda i: (0, i)),
            pl.BlockSpec((E + H, 4 * H), lambda i: (0, 0)),
            pl.BlockSpec((1, 4 * H), lambda i: (0, 0)),
            pl.BlockSpec((B, H), lambda i: (0, 0)),
            pl.BlockSpec((B, H), lambda i: (0, 0)),
        ],
        out_specs=[
            pl.BlockSpec((B, T_BLK * H), lambda i: (0, i)),
            pl.BlockSpec((B, H), lambda i: (0, 0)),
            pl.BlockSpec((B, H), lambda i: (0, 0)),
        ],
        out_shape=[
            jax.ShapeDtypeStruct((B, L * H), jnp.float32),
            jax.ShapeDtypeStruct((B, H), jnp.float32),
            jax.ShapeDtypeStruct((B, H), jnp.float32),
        ],
        compiler_params=pltpu.CompilerParams(
            dimension_semantics=("arbitrary",),
        ),
        interpret=interpret,
    )(x2d, par2d, w_cat, bias, h0, c0)


def kernel(decoder_input, h0, c0, emb, W_ih, W_hh, b_ih, b_hh):
    idx_flat = decoder_input.reshape(-1).astype(jnp.int32)
    idx_p = idx_flat >> 1
    parity = (idx_flat & 1).astype(jnp.int8)
    idx3 = idx_p.reshape(NW, NCHUNK, CHUNK)
    table = jnp.concatenate([emb[0::2], emb[1::2]], axis=1)
    x_flat = _sc_gather(table, idx3)
    x2d = x_flat.reshape(B, L * 2 * E)
    par2d = jnp.broadcast_to(
        parity.reshape(B, L, 1), (B, L, E)
    ).reshape(B, L * E)
    w_cat = jnp.concatenate([W_ih.T, W_hh.T], axis=0)
    bias = (b_ih + b_hh).reshape(1, 4 * H)
    ys2d, h_n, c_n = _lstm(x2d, par2d, w_cat, bias, h0[0], c0[0])
    decoder_output = ys2d.reshape(B, L, H)
    return decoder_output, (h_n[None, :, :], c_n[None, :, :])


# speedup vs baseline: 10.2262x; 10.2262x over previous
import functools

import jax
import jax.numpy as jnp
from jax import lax
from jax.experimental import pallas as pl
from jax.experimental.pallas import tpu as pltpu
from jax.experimental.pallas import tpu_sc as plsc

B = 1024
L = 50
E = 64
H = 64
VHALF = 500000
NW = 32
N_IDX = B * L
B_PER_W = N_IDX // NW
CHUNK = 80
NCHUNK = B_PER_W // CHUNK
NPASS = 2
CPP = NCHUNK // NPASS

T_BLK = 2
N_BLK = L // T_BLK


def _sc_gather_body(table_hbm, idx_hbm, out_hbm, idx_v, rows_v, sem):
    wid = lax.axis_index("s") * 2 + lax.axis_index("c")
    base = wid * B_PER_W
    pltpu.sync_copy(idx_hbm.at[wid], idx_v)
    for p in range(NPASS):
        copies = []
        for j in range(CPP):
            copies.append(
                pltpu.async_copy(
                    table_hbm.at[idx_v.at[p * CPP + j]],
                    rows_v.at[pl.ds(j * CHUNK, CHUNK)],
                    sem,
                )
            )
        for cp in copies:
            cp.wait()
        pltpu.sync_copy(
            rows_v, out_hbm.at[pl.ds(base + p * CPP * CHUNK, CPP * CHUNK)]
        )


def _sc_gather(table, idx3):
    kern = functools.partial(
        pl.kernel,
        mesh=plsc.VectorSubcoreMesh(core_axis_name="c", subcore_axis_name="s"),
        out_type=jax.ShapeDtypeStruct((N_IDX, 2 * E), jnp.float32),
        scratch_types=[
            pltpu.VMEM((NCHUNK, CHUNK), jnp.int32),
            pltpu.VMEM((CPP * CHUNK, 2 * E), jnp.float32),
            pltpu.SemaphoreType.DMA,
        ],
    )(_sc_gather_body)
    return kern(table, idx3)


def _lstm_body(x_ref, sel_ref, w_ref, b_ref, h0_ref, c0_ref,
               ys_ref, h_ref, c_ref):
    i = pl.program_id(0)

    @pl.when(i == 0)
    def _():
        h_ref[...] = h0_ref[...]
        c_ref[...] = c0_ref[...]

    h = h_ref[...]
    c = c_ref[...]
    b = b_ref[...]
    w = w_ref[...]
    for j in range(T_BLK):
        xf = x_ref[j * B:(j + 1) * B, :]
        sel = sel_ref[j * B:(j + 1) * B, :] != 0
        x_t = jnp.where(sel, xf[:, E:2 * E], xf[:, 0:E])
        xh = jnp.concatenate([x_t, h], axis=1)
        gates = jnp.dot(xh, w, preferred_element_type=jnp.float32) + b
        ig = jax.nn.sigmoid(gates[:, 0:H])
        fg = jax.nn.sigmoid(gates[:, H:2 * H])
        gg = jnp.tanh(gates[:, 2 * H:3 * H])
        og = jax.nn.sigmoid(gates[:, 3 * H:4 * H])
        c = fg * c + ig * gg
        h = og * jnp.tanh(c)
        ys_ref[:, j * H:(j + 1) * H] = h
    h_ref[...] = h
    c_ref[...] = c


def _lstm(x, sel2d, w_cat, bias, h0, c0, interpret=False):
    return pl.pallas_call(
        _lstm_body,
        grid=(N_BLK,),
        in_specs=[
            pl.BlockSpec((T_BLK * B, 2 * E), lambda i: (i, 0)),
            pl.BlockSpec((T_BLK * B, E), lambda i: (i, 0)),
            pl.BlockSpec((E + H, 4 * H), lambda i: (0, 0)),
            pl.BlockSpec((1, 4 * H), lambda i: (0, 0)),
            pl.BlockSpec((B, H), lambda i: (0, 0)),
            pl.BlockSpec((B, H), lambda i: (0, 0)),
        ],
        out_specs=[
            pl.BlockSpec((B, T_BLK * H), lambda i: (0, i)),
            pl.BlockSpec((B, H), lambda i: (0, 0)),
            pl.BlockSpec((B, H), lambda i: (0, 0)),
        ],
        out_shape=[
            jax.ShapeDtypeStruct((B, L * H), jnp.float32),
            jax.ShapeDtypeStruct((B, H), jnp.float32),
            jax.ShapeDtypeStruct((B, H), jnp.float32),
        ],
        compiler_params=pltpu.CompilerParams(
            dimension_semantics=("arbitrary",),
        ),
        interpret=interpret,
    )(x, sel2d, w_cat, bias, h0, c0)


def kernel(decoder_input, h0, c0, emb, W_ih, W_hh, b_ih, b_hh):
    idxT = decoder_input.T.reshape(-1).astype(jnp.int32)
    selT = (idxT >= VHALF).astype(jnp.int8)
    idx_p = jnp.where(selT != 0, idxT - VHALF, idxT)
    idx3 = idx_p.reshape(NW, NCHUNK, CHUNK)
    table = jnp.concatenate([emb[:VHALF], emb[VHALF:]], axis=1)
    x = _sc_gather(table, idx3)
    sel2d = jnp.broadcast_to(selT.reshape(N_IDX, 1), (N_IDX, E))
    w_cat = jnp.concatenate([W_ih.T, W_hh.T], axis=0)
    bias = (b_ih + b_hh).reshape(1, 4 * H)
    ys2d, h_n, c_n = _lstm(x, sel2d, w_cat, bias, h0[0], c0[0])
    decoder_output = ys2d.reshape(B, L, H)
    return decoder_output, (h_n[None, :, :], c_n[None, :, :])


# speedup vs baseline: 15.3885x; 1.5048x over previous
import functools

import jax
import jax.numpy as jnp
from jax import lax
from jax.experimental import pallas as pl
from jax.experimental.pallas import tpu as pltpu
from jax.experimental.pallas import tpu_sc as plsc

B = 1024
L = 50
E = 64
H = 64
VHALF = 500000
NW = 32
N_IDX = B * L
B_PER_W = N_IDX // NW
CHUNK = 80
NCHUNK = B_PER_W // CHUNK
NPASS = 2
CPP = NCHUNK // NPASS

T_BLK = 2
N_BLK = L // T_BLK


def _pack_body(in_ref, out_ref):
    x = in_ref[...]
    xT = x.T
    out_ref[...] = jnp.concatenate([xT[0:1024, :], xT[1024:2048, :]], axis=1)


def _pack(embT):
    return pl.pallas_call(
        _pack_body,
        grid=(489,),
        in_specs=[pl.BlockSpec((E, 2048), lambda i: (0, i))],
        out_specs=pl.BlockSpec((1024, 2 * E), lambda i: (i, 0)),
        out_shape=jax.ShapeDtypeStruct((489 * 1024, 2 * E), jnp.float32),
        compiler_params=pltpu.CompilerParams(
            dimension_semantics=("arbitrary",),
        ),
    )(embT)


def _sc_gather_body(table_hbm, idx_hbm, out_hbm, idx_v, rows_v, sem):
    wid = lax.axis_index("s") * 2 + lax.axis_index("c")
    base = wid * B_PER_W
    pltpu.sync_copy(idx_hbm.at[wid], idx_v)
    for p in range(NPASS):
        copies = []
        for j in range(CPP):
            copies.append(
                pltpu.async_copy(
                    table_hbm.at[idx_v.at[p * CPP + j]],
                    rows_v.at[pl.ds(j * CHUNK, CHUNK)],
                    sem,
                )
            )
        for cp in copies:
            cp.wait()
        pltpu.sync_copy(
            rows_v, out_hbm.at[pl.ds(base + p * CPP * CHUNK, CPP * CHUNK)]
        )


def _sc_gather(table, idx3):
    kern = functools.partial(
        pl.kernel,
        mesh=plsc.VectorSubcoreMesh(core_axis_name="c", subcore_axis_name="s"),
        out_type=jax.ShapeDtypeStruct((N_IDX, 2 * E), jnp.float32),
        scratch_types=[
            pltpu.VMEM((NCHUNK, CHUNK), jnp.int32),
            pltpu.VMEM((CPP * CHUNK, 2 * E), jnp.float32),
            pltpu.SemaphoreType.DMA,
        ],
    )(_sc_gather_body)
    return kern(table, idx3)


def _lstm_body(x_ref, sel_ref, w_ref, b_ref, h0_ref, c0_ref,
               ys_ref, h_ref, c_ref):
    i = pl.program_id(0)

    @pl.when(i == 0)
    def _():
        h_ref[...] = h0_ref[...]
        c_ref[...] = c0_ref[...]

    h = h_ref[...]
    c = c_ref[...]
    b = b_ref[...]
    w = w_ref[...]
    for j in range(T_BLK):
        xf = x_ref[j * B:(j + 1) * B, :]
        sel = sel_ref[j * B:(j + 1) * B, :] != 0
        x_t = jnp.where(sel, xf[:, E:2 * E], xf[:, 0:E])
        xh = jnp.concatenate([x_t, h], axis=1)
        gates = jnp.dot(xh, w, preferred_element_type=jnp.float32) + b
        ig = jax.nn.sigmoid(gates[:, 0:H])
        fg = jax.nn.sigmoid(gates[:, H:2 * H])
        gg = jnp.tanh(gates[:, 2 * H:3 * H])
        og = jax.nn.sigmoid(gates[:, 3 * H:4 * H])
        c = fg * c + ig * gg
        h = og * jnp.tanh(c)
        ys_ref[:, j * H:(j + 1) * H] = h
    h_ref[...] = h
    c_ref[...] = c


def _lstm(x, sel2d, w_cat, bias, h0, c0, interpret=False):
    return pl.pallas_call(
        _lstm_body,
        grid=(N_BLK,),
        in_specs=[
            pl.BlockSpec((T_BLK * B, 2 * E), lambda i: (i, 0)),
            pl.BlockSpec((T_BLK * B, E), lambda i: (i, 0)),
            pl.BlockSpec((E + H, 4 * H), lambda i: (0, 0)),
            pl.BlockSpec((1, 4 * H), lambda i: (0, 0)),
            pl.BlockSpec((B, H), lambda i: (0, 0)),
            pl.BlockSpec((B, H), lambda i: (0, 0)),
        ],
        out_specs=[
            pl.BlockSpec((B, T_BLK * H), lambda i: (0, i)),
            pl.BlockSpec((B, H), lambda i: (0, 0)),
            pl.BlockSpec((B, H), lambda i: (0, 0)),
        ],
        out_shape=[
            jax.ShapeDtypeStruct((B, L * H), jnp.float32),
            jax.ShapeDtypeStruct((B, H), jnp.float32),
            jax.ShapeDtypeStruct((B, H), jnp.float32),
        ],
        compiler_params=pltpu.CompilerParams(
            dimension_semantics=("arbitrary",),
        ),
        interpret=interpret,
    )(x, sel2d, w_cat, bias, h0, c0)


def kernel(decoder_input, h0, c0, emb, W_ih, W_hh, b_ih, b_hh):
    idxT = decoder_input.T.reshape(-1).astype(jnp.int32)
    selT = ((idxT >> 10) & 1).astype(jnp.int8)
    idx_p = ((idxT >> 11) << 10) | (idxT & 1023)
    idx3 = idx_p.reshape(NW, NCHUNK, CHUNK)
    table = _pack(emb.T)
    x = _sc_gather(table, idx3)
    sel2d = jnp.broadcast_to(selT.reshape(N_IDX, 1), (N_IDX, E))
    w_cat = jnp.concatenate([W_ih.T, W_hh.T], axis=0)
    bias = (b_ih + b_hh).reshape(1, 4 * H)
    ys2d, h_n, c_n = _lstm(x, sel2d, w_cat, bias, h0[0], c0[0])
    decoder_output = ys2d.reshape(B, L, H)
    return decoder_output, (h_n[None, :, :], c_n[None, :, :])
